# Initial kernel scaffold; baseline (speedup 1.0000x reference)
import functools

import jax
import jax.numpy as jnp
from jax import lax
from jax.experimental import pallas as pl
from jax.experimental.pallas import tpu as pltpu
from jax.experimental.pallas import tpu_sc as plsc

N = 320000
D = 128
B = 16
NW = 32
RPT = N // NW
CH = 2000
NCHUNK = RPT // CH
GRP = CH // 16
WCOLS = 32

_mesh = plsc.VectorSubcoreMesh(core_axis_name="c", subcore_axis_name="s")


@functools.partial(
    pl.kernel,
    mesh=_mesh,
    out_type=jax.ShapeDtypeStruct((NW, 48), jnp.float32),
    scratch_types=[
        pltpu.VMEM((CH, WCOLS), jnp.float32),
        pltpu.VMEM((CH,), jnp.int32),
        pltpu.VMEM((16, 16), jnp.float32),
        pltpu.VMEM((16, 16), jnp.float32),
        pltpu.VMEM((16, 16), jnp.float32),
        pltpu.VMEM((48,), jnp.float32),
    ],
)
def _segsum_kernel(nf_hbm, batch_hbm, out_hbm, fbuf, bbuf, acc9, acc25, accc, obuf):
    wid = lax.axis_index("s") * 2 + lax.axis_index("c")
    base = wid * RPT
    lanes = lax.iota(jnp.int32, 16)
    col9 = jnp.full((16,), 9, jnp.int32)
    col25 = jnp.full((16,), 25, jnp.int32)
    ones = jnp.ones((16,), jnp.float32)
    zeros = jnp.zeros((16,), jnp.float32)

    for l in range(16):
        acc9[l] = zeros
        acc25[l] = zeros
        accc[l] = zeros

    for c in range(NCHUNK):
        r0 = base + c * CH
        pltpu.sync_copy(nf_hbm.at[pl.ds(r0, CH), pl.ds(0, WCOLS)], fbuf)
        pltpu.sync_copy(batch_hbm.at[pl.ds(r0, CH)], bbuf)

        def body(g, carry):
            rows = jnp.full((16,), g * 16, jnp.int32) + lanes
            segv = bbuf[pl.ds(g * 16, 16)]
            v9 = plsc.load_gather(fbuf, [rows, col9])
            v25 = plsc.load_gather(fbuf, [rows, col25])
            plsc.addupdate_scatter(acc9, [lanes, segv], v9)
            plsc.addupdate_scatter(acc25, [lanes, segv], v25)
            plsc.addupdate_scatter(accc, [lanes, segv], ones)
            return carry

        lax.fori_loop(0, GRP, body, 0)

    s9 = zeros
    s25 = zeros
    sc = zeros
    for l in range(16):
        s9 = s9 + acc9[l]
        s25 = s25 + acc25[l]
        sc = sc + accc[l]
    obuf[pl.ds(0, 16)] = sc
    obuf[pl.ds(16, 16)] = s9
    obuf[pl.ds(32, 16)] = s25
    pltpu.sync_copy(obuf, out_hbm.at[wid])


def _mlp_kernel(p_ref, w1_ref, b1_ref, w2_ref, b2_ref, o_ref):
    p = jnp.sum(p_ref[...], axis=0, keepdims=True)
    cnt = p[:, 0:16]
    s9 = p[:, 16:32]
    s25 = p[:, 32:48]
    safe = jnp.maximum(cnt, 1.0)
    r0 = 1.0 - s25 / safe
    r1 = s9 / safe
    h = jnp.maximum(w1_ref[:, 0:1] * r0 + w1_ref[:, 1:2] * r1 + b1_ref[...], 0.0)
    z = jnp.sum(w2_ref[...] * h, axis=0, keepdims=True) + b2_ref[...]
    out = 1.0 / (1.0 + jnp.exp(-z))
    o_ref[...] = jnp.where(cnt > 0.0, out, 0.0)


def kernel(node_features, batch, graph_embedding, W1, b1, W2, b2):
    del graph_embedding
    batch32 = batch.astype(jnp.int32)
    partials = _segsum_kernel(node_features, batch32)
    scores = pl.pallas_call(
        _mlp_kernel,
        out_shape=jax.ShapeDtypeStruct((1, 16), jnp.float32),
    )(partials, W1, b1.reshape(64, 1), W2.reshape(64, 1), b2.reshape(1, 1))
    return scores.reshape(B)


# speedup vs baseline: 18.9141x; 18.9141x over previous
import functools

import jax
import jax.numpy as jnp
from jax import lax
from jax.experimental import pallas as pl
from jax.experimental.pallas import tpu as pltpu
from jax.experimental.pallas import tpu_sc as plsc

N = 320000
D = 128
B = 16
NW = 32
RPT = N // NW
CH = 2000
NCHUNK = RPT // CH
GRP = CH // 16
WCOLS = 32

_mesh = plsc.VectorSubcoreMesh(core_axis_name="c", subcore_axis_name="s")


@functools.partial(
    pl.kernel,
    mesh=_mesh,
    out_type=jax.ShapeDtypeStruct((NW, 48), jnp.float32),
    compiler_params=pltpu.CompilerParams(
        use_tc_tiling_on_sc=False, needs_layout_passes=False
    ),
    scratch_types=[
        pltpu.VMEM((CH, WCOLS), jnp.float32),
        pltpu.VMEM((CH,), jnp.int32),
        pltpu.VMEM((16, 16), jnp.float32),
        pltpu.VMEM((16, 16), jnp.float32),
        pltpu.VMEM((16, 16), jnp.float32),
        pltpu.VMEM((48,), jnp.float32),
    ],
)
def _segsum_kernel(nf_hbm, batch_hbm, out_hbm, fbuf, bbuf, acc9, acc25, accc, obuf):
    wid = lax.axis_index("s") * 2 + lax.axis_index("c")
    base = wid * RPT
    lanes = lax.iota(jnp.int32, 16)
    col9 = jnp.full((16,), 9, jnp.int32)
    col25 = jnp.full((16,), 25, jnp.int32)
    ones = jnp.ones((16,), jnp.float32)
    zeros = jnp.zeros((16,), jnp.float32)

    for l in range(16):
        acc9[l] = zeros
        acc25[l] = zeros
        accc[l] = zeros

    for c in range(NCHUNK):
        r0 = base + c * CH
        pltpu.sync_copy(nf_hbm.at[pl.ds(r0, CH), pl.ds(0, WCOLS)], fbuf)
        pltpu.sync_copy(batch_hbm.at[pl.ds(r0, CH)], bbuf)

        def body(g, carry):
            rows = jnp.full((16,), g * 16, jnp.int32) + lanes
            segv = bbuf[pl.ds(g * 16, 16)]
            v9 = plsc.load_gather(fbuf, [rows, col9])
            v25 = plsc.load_gather(fbuf, [rows, col25])
            plsc.addupdate_scatter(acc9, [lanes, segv], v9)
            plsc.addupdate_scatter(acc25, [lanes, segv], v25)
            plsc.addupdate_scatter(accc, [lanes, segv], ones)
            return carry

        lax.fori_loop(0, GRP, body, 0)

    s9 = zeros
    s25 = zeros
    sc = zeros
    for l in range(16):
        s9 = s9 + acc9[l]
        s25 = s25 + acc25[l]
        sc = sc + accc[l]
    obuf[pl.ds(0, 16)] = sc
    obuf[pl.ds(16, 16)] = s9
    obuf[pl.ds(32, 16)] = s25
    pltpu.sync_copy(obuf, out_hbm.at[wid])


def _mlp_kernel(p_ref, w1_ref, b1_ref, w2_ref, b2_ref, o_ref):
    p = jnp.sum(p_ref[...], axis=0, keepdims=True)
    cnt = p[:, 0:16]
    s9 = p[:, 16:32]
    s25 = p[:, 32:48]
    safe = jnp.maximum(cnt, 1.0)
    r0 = 1.0 - s25 / safe
    r1 = s9 / safe
    h = jnp.maximum(w1_ref[:, 0:1] * r0 + w1_ref[:, 1:2] * r1 + b1_ref[...], 0.0)
    z = jnp.sum(w2_ref[...] * h, axis=0, keepdims=True) + b2_ref[...]
    out = 1.0 / (1.0 + jnp.exp(-z))
    o_ref[...] = jnp.where(cnt > 0.0, out, 0.0)


def kernel(node_features, batch, graph_embedding, W1, b1, W2, b2):
    del graph_embedding
    batch32 = batch.astype(jnp.int32)
    partials = _segsum_kernel(node_features, batch32)
    scores = pl.pallas_call(
        _mlp_kernel,
        out_shape=jax.ShapeDtypeStruct((1, 16), jnp.float32),
    )(partials, W1, b1.reshape(64, 1), W2.reshape(64, 1), b2.reshape(1, 1))
    return scores.reshape(B)


# speedup vs baseline: 22.4603x; 1.1875x over previous
import functools

import jax
import jax.numpy as jnp
from jax import lax
from jax.experimental import pallas as pl
from jax.experimental.pallas import tpu as pltpu
from jax.experimental.pallas import tpu_sc as plsc

N = 320000
D = 128
B = 16
NW = 32
RPT = N // NW
CH = 400
NCHUNK = RPT // CH
GRP = CH // 16
WCOLS = 32

_mesh = plsc.VectorSubcoreMesh(core_axis_name="c", subcore_axis_name="s")


UNROLL = 5


@functools.partial(
    pl.kernel,
    mesh=_mesh,
    out_type=jax.ShapeDtypeStruct((NW, 48), jnp.float32),
    compiler_params=pltpu.CompilerParams(
        use_tc_tiling_on_sc=False, needs_layout_passes=False
    ),
    scratch_types=[
        pltpu.VMEM((2, CH, WCOLS), jnp.float32),
        pltpu.VMEM((2, CH), jnp.int32),
        pltpu.VMEM((16, 16), jnp.float32),
        pltpu.VMEM((16, 16), jnp.float32),
        pltpu.VMEM((16, 16), jnp.float32),
        pltpu.VMEM((48,), jnp.float32),
        pltpu.SemaphoreType.DMA,
        pltpu.SemaphoreType.DMA,
        pltpu.SemaphoreType.DMA,
        pltpu.SemaphoreType.DMA,
    ],
)
def _segsum_kernel(
    nf_hbm, batch_hbm, out_hbm, fbuf, bbuf, acc9, acc25, accc, obuf,
    fsem0, fsem1, bsem0, bsem1,
):
    wid = lax.axis_index("s") * 2 + lax.axis_index("c")
    base = wid * RPT
    lanes = lax.iota(jnp.int32, 16)
    col9 = jnp.full((16,), 9, jnp.int32)
    col25 = jnp.full((16,), 25, jnp.int32)
    ones = jnp.ones((16,), jnp.float32)
    zeros = jnp.zeros((16,), jnp.float32)
    fsems = (fsem0, fsem1)
    bsems = (bsem0, bsem1)

    for l in range(16):
        acc9[l] = zeros
        acc25[l] = zeros
        accc[l] = zeros

    def issue(c):
        slot = c % 2
        r0 = base + c * CH
        fh = pltpu.make_async_copy(
            nf_hbm.at[pl.ds(r0, CH), pl.ds(0, WCOLS)], fbuf.at[slot], fsems[slot]
        )
        fh.start()
        bh = pltpu.make_async_copy(
            batch_hbm.at[pl.ds(r0, CH)], bbuf.at[slot], bsems[slot]
        )
        bh.start()
        return fh, bh

    handles = [None, None]
    handles[0] = issue(0)
    for c in range(NCHUNK):
        slot = c % 2
        if c + 1 < NCHUNK:
            handles[1 - slot] = issue(c + 1)
        fh, bh = handles[slot]
        fh.wait()
        bh.wait()
        fb = fbuf.at[slot]
        bb = bbuf.at[slot]

        def body(i, carry):
            gbase = i * (16 * UNROLL)
            for u in range(UNROLL):
                off = gbase + u * 16
                rows = lanes + off
                segv = bb[pl.ds(off, 16)]
                v9 = plsc.load_gather(fb, [rows, col9])
                v25 = plsc.load_gather(fb, [rows, col25])
                plsc.addupdate_scatter(acc9, [lanes, segv], v9)
                plsc.addupdate_scatter(acc25, [lanes, segv], v25)
                plsc.addupdate_scatter(accc, [lanes, segv], ones)
            return carry

        lax.fori_loop(0, GRP // UNROLL, body, 0)

    s9 = zeros
    s25 = zeros
    sc = zeros
    for l in range(16):
        s9 = s9 + acc9[l]
        s25 = s25 + acc25[l]
        sc = sc + accc[l]
    obuf[pl.ds(0, 16)] = sc
    obuf[pl.ds(16, 16)] = s9
    obuf[pl.ds(32, 16)] = s25
    pltpu.sync_copy(obuf, out_hbm.at[wid])


def _mlp_kernel(p_ref, w1_ref, b1_ref, w2_ref, b2_ref, o_ref):
    p = jnp.sum(p_ref[...], axis=0, keepdims=True)
    cnt = p[:, 0:16]
    s9 = p[:, 16:32]
    s25 = p[:, 32:48]
    safe = jnp.maximum(cnt, 1.0)
    r0 = 1.0 - s25 / safe
    r1 = s9 / safe
    h = jnp.maximum(w1_ref[:, 0:1] * r0 + w1_ref[:, 1:2] * r1 + b1_ref[...], 0.0)
    z = jnp.sum(w2_ref[...] * h, axis=0, keepdims=True) + b2_ref[...]
    out = 1.0 / (1.0 + jnp.exp(-z))
    o_ref[...] = jnp.where(cnt > 0.0, out, 0.0)


def kernel(node_features, batch, graph_embedding, W1, b1, W2, b2):
    del graph_embedding
    batch32 = batch.astype(jnp.int32)
    partials = _segsum_kernel(node_features, batch32)
    scores = pl.pallas_call(
        _mlp_kernel,
        out_shape=jax.ShapeDtypeStruct((1, 16), jnp.float32),
    )(partials, W1, b1.reshape(64, 1), W2.reshape(64, 1), b2.reshape(1, 1))
    return scores.reshape(B)
